# baseline (speedup 1.0000x reference)
import jax
import jax.numpy as jnp
from jax import lax
from jax.experimental import pallas as pl
from jax.experimental.pallas import tpu as pltpu
from jax.experimental.pallas import tpu_sc as plsc

NC = 2
NS = 16
L = 16
NW = NC * NS

B = 128
N = 32768
D = 8
K = 128

ROWS_PER_W = B // NW
CHUNK = 2048
NCHUNK = N // CHUNK
SEG = 256
NSEG = N // SEG
SEGS_PER_CHUNK = CHUNK // SEG
GRP = 128

_BIG = 2**30


def _sc_body(tokgrp_hbm, tokflat_hbm, maskf_hbm, out_tok_hbm, out_msk_hbm,
             keys_v, tok_chunk, msk_chunk, seg_max, idx_v, gidx_v, vals_v,
             msk_out_v, grp_v, rows_v, sem):
    wid = lax.axis_index("s") * NC + lax.axis_index("c")
    lane = lax.iota(jnp.int32, L)
    lane0 = lane == 0
    ch2_lane = lane * D + 2

    def bf(x):
        return jnp.full((L,), x, jnp.float32)

    def bi(x):
        return jnp.full((L,), x, jnp.int32)

    def process_row(r, carry0):
        row = wid * ROWS_PER_W + r

        def chunk_body(c, carry):
            pltpu.sync_copy(
                tokflat_hbm.at[pl.ds((row * N + c * CHUNK) * D, CHUNK * D)],
                tok_chunk)
            pltpu.sync_copy(
                maskf_hbm.at[pl.ds(row * N + c * CHUNK, CHUNK)],
                msk_chunk)

            def seg_body(s, carry1):
                base = s * SEG
                m = bf(-3.0)
                for t in range(SEG // L):
                    off = base + t * L
                    ch2 = plsc.load_gather(tok_chunk, [off * D + ch2_lane])
                    mv = msk_chunk[pl.ds(off, L)]
                    key = jnp.where(mv > 0.5, bf(-1.0), jnp.abs(ch2))
                    keys_v[pl.ds(c * CHUNK + off, L)] = key
                    m = jnp.maximum(m, key)
                plsc.store_scatter(
                    seg_max,
                    [bi(c * SEGS_PER_CHUNK + s)],
                    jnp.full((L,), jnp.max(m)),
                    mask=lane0)
                return carry1

            return lax.fori_loop(0, SEGS_PER_CHUNK, seg_body, carry)

        lax.fori_loop(0, NCHUNK, chunk_body, 0)

        def extract(j, carry):
            m = seg_max[pl.ds(0, L)]
            for i in range(1, NSEG // L):
                m = jnp.maximum(m, seg_max[pl.ds(i * L, L)])
            M = jnp.max(m)

            sstar = jnp.int32(_BIG)
            for i in range(NSEG // L):
                sv = seg_max[pl.ds(i * L, L)]
                cand = jnp.min(jnp.where(sv == M, i * L + lane, _BIG))
                sstar = jnp.minimum(sstar, cand)

            base = sstar * SEG
            eidx = jnp.int32(_BIG)
            for t in range(SEG // L):
                kv = keys_v[pl.ds(base + t * L, L)]
                cand = jnp.min(
                    jnp.where(kv == M, base + t * L + lane, _BIG))
                eidx = jnp.minimum(eidx, cand)

            jslot = bi(j)
            plsc.store_scatter(idx_v, [jslot],
                               jnp.full((L,), N * row + eidx), mask=lane0)
            plsc.store_scatter(vals_v, [jslot], jnp.full((L,), M),
                               mask=lane0)
            plsc.store_scatter(keys_v, [jnp.full((L,), eidx)],
                               bf(-2.0), mask=lane0)

            m2 = keys_v[pl.ds(base, L)]
            for t in range(1, SEG // L):
                m2 = jnp.maximum(m2, keys_v[pl.ds(base + t * L, L)])
            plsc.store_scatter(seg_max, [jnp.full((L,), sstar)],
                               jnp.full((L,), jnp.max(m2)), mask=lane0)
            return carry

        lax.fori_loop(0, K, extract, 0)

        def gi(i, carry):
            t = idx_v[pl.ds(i * L, L)]
            gidx_v[pl.ds(i * L, L)] = t >> 4
            return carry

        lax.fori_loop(0, K // L, gi, 0)
        pltpu.async_copy(tokgrp_hbm.at[gidx_v], grp_v, sem).wait()

        jo = lane >> 3
        do = lane & 7

        def ex(i, carry):
            jv = i * 2 + jo
            tsel = plsc.load_gather(idx_v, [jv])
            col = (tsel & 15) * D + do
            val = plsc.load_gather(grp_v, [jv, col])
            rows_v[pl.ds(i * L, L)] = val
            return carry

        lax.fori_loop(0, K * D // L, ex, 0)
        pltpu.sync_copy(rows_v, out_tok_hbm.at[row])

        def mk(i, carry):
            v = vals_v[pl.ds(i * L, L)]
            msk_out_v[pl.ds(i * L, L)] = jnp.where(
                v == -1.0, bf(1.0), bf(0.0))
            return carry

        lax.fori_loop(0, K // L, mk, 0)
        pltpu.sync_copy(msk_out_v, out_msk_hbm.at[row])
        return carry0

    lax.fori_loop(0, ROWS_PER_W, process_row, 0)


@jax.jit
def _run(tokgrp, tokflat, maskf):
    mesh = plsc.VectorSubcoreMesh(
        core_axis_name="c", subcore_axis_name="s",
        num_cores=NC, num_subcores=NS)
    f = pl.kernel(
        _sc_body,
        out_type=(
            jax.ShapeDtypeStruct((B, K * D), jnp.float32),
            jax.ShapeDtypeStruct((B, K), jnp.float32),
        ),
        mesh=mesh,
        compiler_params=pltpu.CompilerParams(needs_layout_passes=False),
        scratch_types=[
            pltpu.VMEM((N,), jnp.float32),
            pltpu.VMEM((CHUNK * D,), jnp.float32),
            pltpu.VMEM((CHUNK,), jnp.float32),
            pltpu.VMEM((NSEG,), jnp.float32),
            pltpu.VMEM((K,), jnp.int32),
            pltpu.VMEM((K,), jnp.int32),
            pltpu.VMEM((K,), jnp.float32),
            pltpu.VMEM((K,), jnp.float32),
            pltpu.VMEM((K, GRP), jnp.float32),
            pltpu.VMEM((K * D,), jnp.float32),
            pltpu.SemaphoreType.DMA,
        ],
    )
    return f(tokgrp, tokflat, maskf)


def kernel(tokens, obs_mask):
    tokgrp = tokens.reshape(B * N * D // GRP, GRP)
    tokflat = tokens.reshape(B * N * D)
    maskf = obs_mask.reshape(B * N).astype(jnp.float32)
    out_tok, mask_f = _run(tokgrp, tokflat, maskf)
    return out_tok.reshape(B, K, D), mask_f != 0.0


# speedup vs baseline: 7.0556x; 7.0556x over previous
import jax
import jax.numpy as jnp
from jax import lax
from jax.experimental import pallas as pl
from jax.experimental.pallas import tpu as pltpu
from jax.experimental.pallas import tpu_sc as plsc

NC = 2
NS = 16
L = 16
NW = NC * NS

B = 128
N = 32768
D = 8
K = 128

TPG = 128 // D
NG = N // TPG
ROWS_PER_W = B // NW
TPC = 4096
GPC = TPC // TPG
NCH = N // TPC
SEG = 256
NSEG = N // SEG
SEGS_PER_CHUNK = TPC // SEG
LAG = 16

_BIG = 2**30


def _sc_body(tokens_hbm, maskf_hbm, out_tok_hbm, out_msk_hbm,
             keys_v, chunk_v, msk_chunk, seg_max, idx_v, vals_v,
             msk_out_v, wgrp_v, rows_v, idx_s, gsem):
    wid = lax.axis_index("s") * NC + lax.axis_index("c")
    lane = lax.iota(jnp.int32, L)
    lane0 = lane == 0
    ch2_lane = lane * D + 2

    def bf(x):
        return jnp.full((L,), x, jnp.float32)

    def bi(x):
        return jnp.full((L,), x, jnp.int32)

    def process_row(r, carry0):
        row = wid * ROWS_PER_W + r

        def chunk_body(c, carry):
            pltpu.sync_copy(
                tokens_hbm.at[row, pl.ds(c * GPC, GPC), :], chunk_v)
            pltpu.sync_copy(
                maskf_hbm.at[pl.ds(row * N + c * TPC, TPC)], msk_chunk)

            def seg_body(s, carry1):
                base = s * SEG
                m = bf(-3.0)
                for t in range(SEG // L):
                    off = base + t * L
                    ch2 = plsc.load_gather(
                        chunk_v, [bi(off // TPG), ch2_lane])
                    mv = msk_chunk[pl.ds(off, L)]
                    key = jnp.where(mv > 0.5, bf(-1.0), jnp.abs(ch2))
                    keys_v[pl.ds(c * TPC + off, L)] = key
                    m = jnp.maximum(m, key)
                plsc.store_scatter(
                    seg_max,
                    [bi(c * SEGS_PER_CHUNK + s)],
                    jnp.full((L,), jnp.max(m)),
                    mask=lane0)
                return carry1

            return lax.fori_loop(0, SEGS_PER_CHUNK, seg_body, carry)

        lax.fori_loop(0, NCH, chunk_body, 0)

        def extract(j, carry):
            m = seg_max[pl.ds(0, L)]
            for i in range(1, NSEG // L):
                m = jnp.maximum(m, seg_max[pl.ds(i * L, L)])
            M = jnp.max(m)

            sstar = jnp.int32(_BIG)
            for i in range(NSEG // L):
                sv = seg_max[pl.ds(i * L, L)]
                cand = jnp.min(jnp.where(sv == M, i * L + lane, _BIG))
                sstar = jnp.minimum(sstar, cand)

            base = sstar * SEG
            eidx = jnp.int32(_BIG)
            for t in range(SEG // L):
                kv = keys_v[pl.ds(base + t * L, L)]
                cand = jnp.min(
                    jnp.where(kv == M, base + t * L + lane, _BIG))
                eidx = jnp.minimum(eidx, cand)

            idx_s[j] = eidx
            plsc.store_scatter(idx_v, [bi(j)], jnp.full((L,), eidx),
                               mask=lane0)
            plsc.store_scatter(vals_v, [bi(j)], jnp.full((L,), M),
                               mask=lane0)
            plsc.store_scatter(keys_v, [jnp.full((L,), eidx)],
                               bf(-2.0), mask=lane0)

            m2 = keys_v[pl.ds(base, L)]
            for t in range(1, SEG // L):
                m2 = jnp.maximum(m2, keys_v[pl.ds(base + t * L, L)])
            plsc.store_scatter(seg_max, [jnp.full((L,), sstar)],
                               jnp.full((L,), jnp.max(m2)), mask=lane0)
            return carry

        lax.fori_loop(0, K, extract, 0)

        def fetch(j, carry):
            @pl.when(j < K)
            def _():
                e = idx_s[j]
                pltpu.async_copy(
                    tokens_hbm.at[row, pl.ds(e // TPG, 1), :],
                    wgrp_v.at[pl.ds(j, 1), :], gsem)

            @pl.when(j >= LAG)
            def _():
                pltpu.make_async_copy(
                    tokens_hbm.at[row, pl.ds(0, 1), :],
                    wgrp_v.at[pl.ds(0, 1), :], gsem).wait()

            return carry

        lax.fori_loop(0, K + LAG, fetch, 0)

        jo = lane >> 3
        do = lane & 7

        def ex(i, carry):
            jv = i * 2 + jo
            tsel = plsc.load_gather(idx_v, [jv])
            col = (tsel & (TPG - 1)) * D + do
            rows_v[pl.ds(i * L, L)] = plsc.load_gather(wgrp_v, [jv, col])
            return carry

        lax.fori_loop(0, K * D // L, ex, 0)
        pltpu.sync_copy(rows_v, out_tok_hbm.at[row])

        def mk(i, carry):
            v = vals_v[pl.ds(i * L, L)]
            msk_out_v[pl.ds(i * L, L)] = jnp.where(
                v == -1.0, bf(1.0), bf(0.0))
            return carry

        lax.fori_loop(0, K // L, mk, 0)
        pltpu.sync_copy(msk_out_v, out_msk_hbm.at[row])
        return carry0

    lax.fori_loop(0, ROWS_PER_W, process_row, 0)


@jax.jit
def _run(tokens3, maskf):
    mesh = plsc.VectorSubcoreMesh(
        core_axis_name="c", subcore_axis_name="s",
        num_cores=NC, num_subcores=NS)
    f = pl.kernel(
        _sc_body,
        out_type=(
            jax.ShapeDtypeStruct((B, K * D), jnp.float32),
            jax.ShapeDtypeStruct((B, K), jnp.float32),
        ),
        mesh=mesh,
        compiler_params=pltpu.CompilerParams(needs_layout_passes=False),
        scratch_types=[
            pltpu.VMEM((N,), jnp.float32),
            pltpu.VMEM((GPC, 128), jnp.float32),
            pltpu.VMEM((TPC,), jnp.float32),
            pltpu.VMEM((NSEG,), jnp.float32),
            pltpu.VMEM((K,), jnp.int32),
            pltpu.VMEM((K,), jnp.float32),
            pltpu.VMEM((K,), jnp.float32),
            pltpu.VMEM((K, 128), jnp.float32),
            pltpu.VMEM((K * D,), jnp.float32),
            pltpu.SMEM((K,), jnp.int32),
            pltpu.SemaphoreType.DMA,
        ],
    )
    return f(tokens3, maskf)


def kernel(tokens, obs_mask):
    tokens3 = tokens.reshape(B, NG, 128)
    maskf = obs_mask.reshape(B * N).astype(jnp.float32)
    out_tok, mask_f = _run(tokens3, maskf)
    return out_tok.reshape(B, K, D), mask_f != 0.0


# speedup vs baseline: 7.1648x; 1.0155x over previous
import jax
import jax.numpy as jnp
from jax import lax
from jax.experimental import pallas as pl
from jax.experimental.pallas import tpu as pltpu
from jax.experimental.pallas import tpu_sc as plsc

NC = 2
NS = 16
L = 16
NW = NC * NS

B = 128
N = 32768
D = 8
K = 128

TPG = 128 // D
NG = N // TPG
ROWS_PER_W = B // NW
TPC = 4096
GPC = TPC // TPG
NCH = N // TPC
SEG = 256
NSEG = N // SEG
SEGS_PER_CHUNK = TPC // SEG
LAG = 16

_BIG = 2**30


def _sc_body(tokens_hbm, maskf_hbm, out_tok_hbm, out_msk_hbm,
             keys_v, chunk_v, msk_chunk, seg_max, idx_v, vals_v,
             msk_out_v, wgrp_v, rows_v, idx_s, gsem):
    wid = lax.axis_index("s") * NC + lax.axis_index("c")
    lane = lax.iota(jnp.int32, L)
    lane0 = lane == 0
    ch2_lane = lane * D + 2

    def bf(x):
        return jnp.full((L,), x, jnp.float32)

    def bi(x):
        return jnp.full((L,), x, jnp.int32)

    def process_row(r, carry0):
        row = wid * ROWS_PER_W + r

        def chunk_body(c, carry):
            pltpu.sync_copy(
                tokens_hbm.at[row, pl.ds(c * GPC, GPC), :], chunk_v)
            pltpu.sync_copy(
                maskf_hbm.at[row, pl.ds(c * TPC, TPC)], msk_chunk)

            def seg_body(s, carry1):
                base = s * SEG
                m = bf(-3.0)
                for t in range(SEG // L):
                    off = base + t * L
                    ch2 = plsc.load_gather(
                        chunk_v, [bi(off // TPG), ch2_lane])
                    mv = msk_chunk[pl.ds(off, L)]
                    key = jnp.where(mv > 0.5, bf(-1.0), jnp.abs(ch2))
                    keys_v[pl.ds(c * TPC + off, L)] = key
                    m = jnp.maximum(m, key)
                plsc.store_scatter(
                    seg_max,
                    [bi(c * SEGS_PER_CHUNK + s)],
                    jnp.full((L,), jnp.max(m)),
                    mask=lane0)
                return carry1

            return lax.fori_loop(0, SEGS_PER_CHUNK, seg_body, carry)

        lax.fori_loop(0, NCH, chunk_body, 0)

        def extract(j, carry):
            m = seg_max[pl.ds(0, L)]
            for i in range(1, NSEG // L):
                m = jnp.maximum(m, seg_max[pl.ds(i * L, L)])
            M = jnp.max(m)

            sstar = jnp.int32(_BIG)
            for i in range(NSEG // L):
                sv = seg_max[pl.ds(i * L, L)]
                cand = jnp.min(jnp.where(sv == M, i * L + lane, _BIG))
                sstar = jnp.minimum(sstar, cand)

            base = sstar * SEG
            eidx = jnp.int32(_BIG)
            for t in range(SEG // L):
                kv = keys_v[pl.ds(base + t * L, L)]
                cand = jnp.min(
                    jnp.where(kv == M, base + t * L + lane, _BIG))
                eidx = jnp.minimum(eidx, cand)

            idx_s[j] = eidx
            plsc.store_scatter(idx_v, [bi(j)], jnp.full((L,), eidx),
                               mask=lane0)
            plsc.store_scatter(vals_v, [bi(j)], jnp.full((L,), M),
                               mask=lane0)
            plsc.store_scatter(keys_v, [jnp.full((L,), eidx)],
                               bf(-2.0), mask=lane0)

            m2 = keys_v[pl.ds(base, L)]
            for t in range(1, SEG // L):
                m2 = jnp.maximum(m2, keys_v[pl.ds(base + t * L, L)])
            plsc.store_scatter(seg_max, [jnp.full((L,), sstar)],
                               jnp.full((L,), jnp.max(m2)), mask=lane0)
            return carry

        lax.fori_loop(0, K, extract, 0)

        def fetch(j, carry):
            @pl.when(j < K)
            def _():
                e = idx_s[j]
                pltpu.async_copy(
                    tokens_hbm.at[row, pl.ds(e // TPG, 1), :],
                    wgrp_v.at[pl.ds(j, 1), :], gsem)

            @pl.when(j >= LAG)
            def _():
                pltpu.make_async_copy(
                    tokens_hbm.at[row, pl.ds(0, 1), :],
                    wgrp_v.at[pl.ds(0, 1), :], gsem).wait()

            return carry

        lax.fori_loop(0, K + LAG, fetch, 0)

        jo = lane >> 3
        do = lane & 7

        def ex(i, carry):
            jv = i * 2 + jo
            tsel = plsc.load_gather(idx_v, [jv])
            col = (tsel & (TPG - 1)) * D + do
            rows_v[pl.ds(i * L, L)] = plsc.load_gather(wgrp_v, [jv, col])
            return carry

        lax.fori_loop(0, K * D // L, ex, 0)
        pltpu.sync_copy(rows_v, out_tok_hbm.at[row])

        def mk(i, carry):
            v = vals_v[pl.ds(i * L, L)]
            msk_out_v[pl.ds(i * L, L)] = jnp.where(
                v == -1.0, bf(1.0), bf(0.0))
            return carry

        lax.fori_loop(0, K // L, mk, 0)
        pltpu.sync_copy(msk_out_v, out_msk_hbm.at[row])
        return carry0

    lax.fori_loop(0, ROWS_PER_W, process_row, 0)


@jax.jit
def _run(tokens3, maskf):
    mesh = plsc.VectorSubcoreMesh(
        core_axis_name="c", subcore_axis_name="s",
        num_cores=NC, num_subcores=NS)
    f = pl.kernel(
        _sc_body,
        out_type=(
            jax.ShapeDtypeStruct((B, K * D), jnp.float32),
            jax.ShapeDtypeStruct((B, K), jnp.float32),
        ),
        mesh=mesh,
        compiler_params=pltpu.CompilerParams(needs_layout_passes=False),
        scratch_types=[
            pltpu.VMEM((N,), jnp.float32),
            pltpu.VMEM((GPC, 128), jnp.float32),
            pltpu.VMEM((TPC,), jnp.float32),
            pltpu.VMEM((NSEG,), jnp.float32),
            pltpu.VMEM((K,), jnp.int32),
            pltpu.VMEM((K,), jnp.float32),
            pltpu.VMEM((K,), jnp.float32),
            pltpu.VMEM((K, 128), jnp.float32),
            pltpu.VMEM((K * D,), jnp.float32),
            pltpu.SMEM((K,), jnp.int32),
            pltpu.SemaphoreType.DMA,
        ],
    )
    return f(tokens3, maskf)


def kernel(tokens, obs_mask):
    tokens3 = tokens.reshape(B, NG, 128)
    maskf = obs_mask.astype(jnp.float32)
    out_tok, mask_f = _run(tokens3, maskf)
    return out_tok.reshape(B, K, D), mask_f != 0.0
